# baseline (speedup 1.0000x reference)
import functools

import jax
import jax.numpy as jnp
from jax import lax
from jax.experimental import pallas as pl
from jax.experimental.pallas import tpu as pltpu
from jax.experimental.pallas import tpu_sc as plsc

D = 32
IPG = 128
K = 10
CHUNK = K * IPG


@functools.cache
def _build_gather(total, nc, ns):
    nw = nc * ns
    per_w = total // nw
    rows_per_w = per_w // IPG
    nch = per_w // CHUNK
    assert nch % 2 == 0 and nch >= 4 and nch * CHUNK == per_w

    mesh = plsc.VectorSubcoreMesh(
        core_axis_name="c", subcore_axis_name="s",
        num_cores=nc, num_subcores=ns)

    @functools.partial(
        pl.kernel,
        out_type=jax.ShapeDtypeStruct((total, D), jnp.float32),
        mesh=mesh,
        scratch_types=[
            pltpu.VMEM((2, K, IPG), jnp.int32),
            pltpu.VMEM((2, CHUNK, D), jnp.float32),
            pltpu.SemaphoreType.DMA,
            pltpu.SemaphoreType.DMA,
            pltpu.SemaphoreType.DMA,
            pltpu.SemaphoreType.DMA,
            pltpu.SemaphoreType.DMA,
            pltpu.SemaphoreType.DMA,
        ],
        compiler_params=pltpu.CompilerParams(use_tc_tiling_on_sc=False),
    )
    def gather_kernel(table_hbm, idx_hbm, out_hbm, idx_v, rows_v,
                      isem0, isem1, gsem0, gsem1, osem0, osem1):
        wid = lax.axis_index("s") * nc + lax.axis_index("c")
        row0 = wid * rows_per_w
        base = wid * per_w
        isem = (isem0, isem1)
        gsem = (gsem0, gsem1)
        osem = (osem0, osem1)

        def s_idx(c, b):
            pltpu.make_async_copy(
                idx_hbm.at[pl.ds(row0 + c * K, K)], idx_v.at[b], isem[b]
            ).start()

        def w_idx(b):
            pltpu.make_async_copy(
                idx_hbm.at[pl.ds(row0, K)], idx_v.at[b], isem[b]
            ).wait()

        def g_fire(b):
            for j in range(K):
                pltpu.make_async_copy(
                    table_hbm.at[idx_v.at[b, j]],
                    rows_v.at[b, pl.ds(j * IPG, IPG)],
                    gsem[b],
                ).start()

        def g_drain(b):
            for j in range(K):
                pltpu.make_async_copy(
                    table_hbm.at[idx_v.at[b, j]],
                    rows_v.at[b, pl.ds(j * IPG, IPG)],
                    gsem[b],
                ).wait()

        def s_out(c, b):
            pltpu.make_async_copy(
                rows_v.at[b], out_hbm.at[pl.ds(base + c * CHUNK, CHUNK)],
                osem[b],
            ).start()

        def w_out(b):
            pltpu.make_async_copy(
                rows_v.at[b], out_hbm.at[pl.ds(base, CHUNK)], osem[b]
            ).wait()

        s_idx(0, 0)
        s_idx(1, 1)
        w_idx(0)
        g_fire(0)
        w_idx(1)
        g_fire(1)
        g_drain(0)
        s_out(0, 0)
        s_idx(2, 0)

        def pair(p, carry):
            cA = 2 * p + 1
            w_idx(0)
            w_out(0)
            g_fire(0)
            g_drain(1)
            s_out(cA, 1)

            @pl.when(cA + 2 < nch)
            def _():
                s_idx(cA + 2, 1)

            cB = cA + 1
            w_idx(1)
            w_out(1)
            g_fire(1)
            g_drain(0)
            s_out(cB, 0)

            @pl.when(cB + 2 < nch)
            def _():
                s_idx(cB + 2, 0)

            return carry

        lax.fori_loop(0, (nch - 2) // 2, pair, 0)

        g_drain(1)
        s_out(nch - 1, 1)
        w_out(0)
        w_out(1)

    return gather_kernel


def _transpose_body(in_ref, out_ref):
    eye = jnp.eye(128, dtype=jnp.float32)

    def bt_step(bt, carry):
        x = in_ref[0, pl.ds(bt * 128, 128), :]
        xt = jax.lax.dot_general(
            x, eye, (((0,), (0,)), ((), ())),
            preferred_element_type=jnp.float32)
        out_ref[0, :, bt] = xt.reshape(4, 8, 128)
        return carry

    lax.fori_loop(0, in_ref.shape[1] // 128, bt_step, 0)


@functools.cache
def _build_transpose(l_dim, bt_dim):
    return pl.pallas_call(
        _transpose_body,
        grid=(l_dim,),
        in_specs=[pl.BlockSpec((1, bt_dim * 128, D), lambda l: (l, 0, 0))],
        out_specs=pl.BlockSpec((1, D // 8, bt_dim, 8, 128),
                               lambda l: (l, 0, 0, 0, 0)),
        out_shape=jax.ShapeDtypeStruct((l_dim, D // 8, bt_dim, 8, 128),
                                       jnp.float32),
    )


def kernel(actions, table):
    b, l = actions.shape
    total = b * l
    bt_dim = b // 128
    info = plsc.get_sparse_core_info()
    nc, ns = info.num_cores, info.num_subcores
    idx2d = actions.astype(jnp.int32).T.reshape(total // IPG, IPG)
    interm = _build_gather(total, nc, ns)(table, idx2d)
    out5 = _build_transpose(l, bt_dim)(interm.reshape(l, b, D))
    return out5.transpose(2, 4, 0, 1, 3).reshape(b, l, 1, D)


# speedup vs baseline: 2.4713x; 2.4713x over previous
import functools

import jax
import jax.numpy as jnp
from jax import lax
from jax.experimental import pallas as pl
from jax.experimental.pallas import tpu as pltpu
from jax.experimental.pallas import tpu_sc as plsc

D = 32
IPG = 128
BT_PER_TASK = 4
TASK = BT_PER_TASK * IPG


@functools.cache
def _build(l_dim, b_dim, nc, ns):
    total = l_dim * b_dim
    nw = nc * ns
    ntask = total // TASK
    tpw = ntask // nw
    assert tpw % 2 == 0 and tpw >= 4 and ntask * TASK == total
    g_rows = total // IPG
    out_rows = total * D // (8 * IPG)

    mesh = plsc.VectorSubcoreMesh(
        core_axis_name="c", subcore_axis_name="s",
        num_cores=nc, num_subcores=ns)

    @functools.partial(
        pl.kernel,
        out_type=jax.ShapeDtypeStruct((out_rows, 8, IPG), jnp.float32),
        mesh=mesh,
        scratch_types=[
            pltpu.VMEM((2, BT_PER_TASK, IPG), jnp.int32),
            pltpu.VMEM((2, TASK, D), jnp.float32),
            pltpu.VMEM((2, D // 8, BT_PER_TASK, 8, IPG), jnp.float32),
            pltpu.SemaphoreType.DMA,
            pltpu.SemaphoreType.DMA,
            pltpu.SemaphoreType.DMA,
            pltpu.SemaphoreType.DMA,
            pltpu.SemaphoreType.DMA,
            pltpu.SemaphoreType.DMA,
        ],
        compiler_params=pltpu.CompilerParams(
            use_tc_tiling_on_sc=False, needs_layout_passes=False),
    )
    def gather_kernel(table_hbm, idx_hbm, out_hbm, idx_v, rows_v, stg_v,
                      isem0, isem1, gsem0, gsem1, osem0, osem1):
        wid = lax.axis_index("s") * nc + lax.axis_index("c")
        t0 = wid * tpw
        isem = (isem0, isem1)
        gsem = (gsem0, gsem1)
        osem = (osem0, osem1)
        lanes = lax.iota(jnp.int32, 16)

        def s_idx(t, bf):
            pltpu.make_async_copy(
                idx_hbm.at[pl.ds(t * BT_PER_TASK, BT_PER_TASK)],
                idx_v.at[bf], isem[bf],
            ).start()

        def w_idx(bf):
            pltpu.make_async_copy(
                idx_hbm.at[pl.ds(0, BT_PER_TASK)], idx_v.at[bf], isem[bf]
            ).wait()

        def g_fire(bf):
            for j in range(BT_PER_TASK):
                pltpu.make_async_copy(
                    table_hbm.at[idx_v.at[bf, j]],
                    rows_v.at[bf, pl.ds(j * IPG, IPG)],
                    gsem[bf],
                ).start()

        def g_drain(bf):
            for j in range(BT_PER_TASK):
                pltpu.make_async_copy(
                    table_hbm.at[idx_v.at[bf, j]],
                    rows_v.at[bf, pl.ds(j * IPG, IPG)],
                    gsem[bf],
                ).wait()

        def transpose(bf):
            src = rows_v.at[bf]

            def tile_step(k, carry):
                dt = k // BT_PER_TASK
                btl = k % BT_PER_TASK
                for dr in range(8):
                    d = dt * 8 + dr
                    col = jnp.broadcast_to(d, (16,)).astype(jnp.int32)
                    for q in range(IPG // 16):
                        row = btl * IPG + q * 16 + lanes
                        v = plsc.load_gather(src, [row, col])
                        stg_v[bf, dt, btl, dr, pl.ds(q * 16, 16)] = v
                return carry

            lax.fori_loop(0, (D // 8) * BT_PER_TASK, tile_step, 0)

        def s_out(t, bf):
            l = t // (b_dim // TASK)
            btg = t % (b_dim // TASK)
            for dt in range(D // 8):
                row0 = l * (b_dim // IPG) * (D // 8) + dt * (b_dim // IPG) \
                    + btg * BT_PER_TASK
                pltpu.make_async_copy(
                    stg_v.at[bf, dt],
                    out_hbm.at[pl.ds(row0, BT_PER_TASK)],
                    osem[bf],
                ).start()

        def w_out(bf):
            for dt in range(D // 8):
                pltpu.make_async_copy(
                    stg_v.at[bf, dt], out_hbm.at[pl.ds(0, BT_PER_TASK)],
                    osem[bf],
                ).wait()

        s_idx(t0, 0)
        s_idx(t0 + 1, 1)
        w_idx(0)
        g_fire(0)

        def step(i, bf):
            t = t0 + i
            g_drain(bf)

            @pl.when(i + 1 < tpw)
            def _():
                w_idx(1 - bf)
                g_fire(1 - bf)

            @pl.when(i + 2 < tpw)
            def _():
                s_idx(t + 2, bf)

            @pl.when(i >= 2)
            def _():
                w_out(bf)

            transpose(bf)
            s_out(t, bf)

        def pair(p, carry):
            step(2 * p, 0)
            step(2 * p + 1, 1)
            return carry

        lax.fori_loop(0, tpw // 2, pair, 0)
        w_out(0)
        w_out(1)

    return gather_kernel


def kernel(actions, table):
    b, l = actions.shape
    info = plsc.get_sparse_core_info()
    nc, ns = info.num_cores, info.num_subcores
    idx2d = actions.astype(jnp.int32).T.reshape((b * l) // IPG, IPG)
    res = _build(l, b, nc, ns)(table, idx2d)
    out5 = res.reshape(l, D // 8, b // IPG, 8, IPG)
    return out5.transpose(2, 4, 0, 1, 3).reshape(b, l, 1, D)


# speedup vs baseline: 3.3909x; 1.3721x over previous
import functools

import jax
import jax.numpy as jnp
from jax import lax
from jax.experimental import pallas as pl
from jax.experimental.pallas import tpu as pltpu
from jax.experimental.pallas import tpu_sc as plsc

D = 32
IPG = 128
BT_PER_TASK = 4
TASK = BT_PER_TASK * IPG


@functools.cache
def _build(l_dim, b_dim, nc, ns):
    total = l_dim * b_dim
    nw = nc * ns
    ntask = total // TASK
    tpw = ntask // nw
    assert tpw % 2 == 0 and tpw >= 4 and ntask * TASK == total
    g_rows = total // IPG
    out_rows = total * D // (8 * IPG)

    mesh = plsc.VectorSubcoreMesh(
        core_axis_name="c", subcore_axis_name="s",
        num_cores=nc, num_subcores=ns)

    @functools.partial(
        pl.kernel,
        out_type=jax.ShapeDtypeStruct((out_rows, 8, IPG), jnp.float32),
        mesh=mesh,
        scratch_types=[
            pltpu.VMEM((2, BT_PER_TASK, IPG), jnp.int32),
            pltpu.VMEM((2, TASK, D), jnp.float32),
            pltpu.VMEM((2, D // 8, BT_PER_TASK, 8, IPG), jnp.float32),
            pltpu.SemaphoreType.DMA,
            pltpu.SemaphoreType.DMA,
            pltpu.SemaphoreType.DMA,
            pltpu.SemaphoreType.DMA,
            pltpu.SemaphoreType.DMA,
            pltpu.SemaphoreType.DMA,
        ],
        compiler_params=pltpu.CompilerParams(
            use_tc_tiling_on_sc=False, needs_layout_passes=False),
    )
    def gather_kernel(table_hbm, idx_hbm, out_hbm, idx_v, rows_v, stg_v,
                      isem0, isem1, gsem0, gsem1, osem0, osem1):
        wid = lax.axis_index("s") * nc + lax.axis_index("c")
        t0 = wid * tpw
        isem = (isem0, isem1)
        gsem = (gsem0, gsem1)
        osem = (osem0, osem1)
        lanes = lax.iota(jnp.int32, 16)

        def s_idx(t, bf):
            pltpu.make_async_copy(
                idx_hbm.at[pl.ds(t * BT_PER_TASK, BT_PER_TASK)],
                idx_v.at[bf], isem[bf],
            ).start()

        def w_idx(bf):
            pltpu.make_async_copy(
                idx_hbm.at[pl.ds(0, BT_PER_TASK)], idx_v.at[bf], isem[bf]
            ).wait()

        def g_fire(bf):
            for j in range(BT_PER_TASK):
                pltpu.make_async_copy(
                    table_hbm.at[idx_v.at[bf, j]],
                    rows_v.at[bf, pl.ds(j * IPG, IPG)],
                    gsem[bf],
                ).start()

        def g_drain(bf):
            for j in range(BT_PER_TASK):
                pltpu.make_async_copy(
                    table_hbm.at[idx_v.at[bf, j]],
                    rows_v.at[bf, pl.ds(j * IPG, IPG)],
                    gsem[bf],
                ).wait()

        def transpose(bf):
            src = rows_v.at[bf]

            @plsc.parallel_loop(0, (D // 8) * BT_PER_TASK, 1, unroll=2)
            def _(k):
                dt = k // BT_PER_TASK
                btl = k % BT_PER_TASK
                for dr in range(8):
                    d = dt * 8 + dr
                    col = jnp.broadcast_to(d, (16,)).astype(jnp.int32)
                    for q in range(IPG // 16):
                        row = btl * IPG + q * 16 + lanes
                        v = plsc.load_gather(src, [row, col])
                        stg_v[bf, dt, btl, dr, pl.ds(q * 16, 16)] = v

        def s_out(t, bf):
            l = t // (b_dim // TASK)
            btg = t % (b_dim // TASK)
            for dt in range(D // 8):
                row0 = l * (b_dim // IPG) * (D // 8) + dt * (b_dim // IPG) \
                    + btg * BT_PER_TASK
                pltpu.make_async_copy(
                    stg_v.at[bf, dt],
                    out_hbm.at[pl.ds(row0, BT_PER_TASK)],
                    osem[bf],
                ).start()

        def w_out(bf):
            for dt in range(D // 8):
                pltpu.make_async_copy(
                    stg_v.at[bf, dt], out_hbm.at[pl.ds(0, BT_PER_TASK)],
                    osem[bf],
                ).wait()

        s_idx(t0, 0)
        s_idx(t0 + 1, 1)
        w_idx(0)
        g_fire(0)

        def step(i, bf):
            t = t0 + i
            g_drain(bf)

            @pl.when(i + 1 < tpw)
            def _():
                w_idx(1 - bf)
                g_fire(1 - bf)

            @pl.when(i + 2 < tpw)
            def _():
                s_idx(t + 2, bf)

            @pl.when(i >= 2)
            def _():
                w_out(bf)

            transpose(bf)
            s_out(t, bf)

        def pair(p, carry):
            step(2 * p, 0)
            step(2 * p + 1, 1)
            return carry

        lax.fori_loop(0, tpw // 2, pair, 0)
        w_out(0)
        w_out(1)

    return gather_kernel


def kernel(actions, table):
    b, l = actions.shape
    info = plsc.get_sparse_core_info()
    nc, ns = info.num_cores, info.num_subcores
    idx2d = actions.astype(jnp.int32).T.reshape((b * l) // IPG, IPG)
    res = _build(l, b, nc, ns)(table, idx2d)
    out5 = res.reshape(l, D // 8, b // IPG, 8, IPG)
    return out5.transpose(2, 4, 0, 1, 3).reshape(b, l, 1, D)
